# baseline (speedup 1.0000x reference)
import functools

import jax
import jax.numpy as jnp
from jax import lax
from jax.experimental import pallas as pl
from jax.experimental.pallas import tpu as pltpu
from jax.experimental.pallas import tpu_sc as plsc

BATCH = 4096
DIM = 10000
LANES = 16
NUM_WORKERS = 32
NSEG = DIM // LANES
G = 8
NGROUPS = DIM // G
MAXP = NGROUPS // NUM_WORKERS + 1
TSTEPS = (MAXP + 2 + 2) // 3


def _sc_permute(inp_t, sel):
    mesh = plsc.VectorSubcoreMesh(core_axis_name="c", subcore_axis_name="s")

    @functools.partial(
        pl.kernel,
        mesh=mesh,
        out_type=jax.ShapeDtypeStruct((DIM, BATCH), jnp.float32),
        scratch_types=[
            pltpu.VMEM((DIM,), jnp.int32),
            pltpu.VMEM((DIM,), jnp.int32),
            pltpu.VMEM((G, BATCH), jnp.float32),
            pltpu.VMEM((G, BATCH), jnp.float32),
            pltpu.SemaphoreType.DMA,
            pltpu.SemaphoreType.DMA,
            pltpu.SemaphoreType.DMA,
            pltpu.SemaphoreType.DMA,
        ],
        compiler_params=pltpu.CompilerParams(needs_layout_passes=False),
    )
    def k(in_hbm, sel_hbm, out_hbm, sel_v, c_v, b0, b1, g0, g1, w0, w1):
        cid = lax.axis_index("c")
        sid = lax.axis_index("s")
        wid = sid * 2 + cid
        nvalid = jnp.where(wid < NGROUPS % NUM_WORKERS, MAXP, MAXP - 1)

        pltpu.sync_copy(sel_hbm, sel_v)

        @plsc.parallel_loop(0, NSEG, unroll=8)
        def _(j):
            off = pl.multiple_of(j * LANES, LANES)
            seg = sel_v[pl.ds(off, LANES)]
            c_v[pl.ds(off, LANES)] = plsc.load_gather(sel_v, [seg])

        def fire_gather(p, buf, gsem):
            grp = wid + NUM_WORKERS * p
            idx = c_v.at[pl.ds(pl.multiple_of(grp * G, G), G)]
            pltpu.async_copy(in_hbm.at[idx], buf, gsem)

        fire_gather(0, b0, g0)

        def step(t, carry):
            for par, buf, gsem, obuf, ogsem, owsem, wsem in (
                (0, b0, g0, b1, g1, w1, w0),
                (1, b1, g1, b0, g0, w0, w1),
            ):
                p = 2 * t + par
                grp = wid + NUM_WORKERS * p

                @pl.when((p + 1 < nvalid) & (p >= 1))
                def _():
                    pltpu.make_async_copy(obuf, out_hbm.at[pl.ds(0, G)],
                                          owsem).wait()

                @pl.when(p + 1 < nvalid)
                def _():
                    fire_gather(p + 1, obuf, ogsem)

                @pl.when(p < nvalid)
                def _():
                    pltpu.make_async_copy(in_hbm.at[pl.ds(0, G)], buf,
                                          gsem).wait()
                    pltpu.async_copy(
                        buf, out_hbm.at[pl.ds(pl.multiple_of(grp * G, G), G)],
                        wsem)

            return carry

        lax.fori_loop(0, MAXP // 2, step, 0)

        pltpu.make_async_copy(b0, out_hbm.at[pl.ds(0, G)], w0).wait()
        pltpu.make_async_copy(b1, out_hbm.at[pl.ds(0, G)], w1).wait()

    return k(inp_t, sel)


def kernel(input, forward_indices, backward_indices, shifts):
    ident = jnp.arange(DIM, dtype=jnp.int32)
    sel = jnp.where(
        shifts > 0,
        forward_indices,
        jnp.where(shifts < 0, backward_indices, ident),
    )
    out_t = _sc_permute(jnp.transpose(input), sel)
    return jnp.transpose(out_t)


# speedup vs baseline: 1.0019x; 1.0019x over previous
import functools

import jax
import jax.numpy as jnp
from jax import lax
from jax.experimental import pallas as pl
from jax.experimental.pallas import tpu as pltpu
from jax.experimental.pallas import tpu_sc as plsc

BATCH = 4096
DIM = 10000
LANES = 16
NUM_WORKERS = 32
NSEG = DIM // LANES
G = 8
NGROUPS = DIM // G
MAXP = NGROUPS // NUM_WORKERS + 1
TSTEPS = (MAXP + 2 + 2) // 3


def _sc_permute(inp_t, sel):
    mesh = plsc.VectorSubcoreMesh(core_axis_name="c", subcore_axis_name="s")

    @functools.partial(
        pl.kernel,
        mesh=mesh,
        out_type=jax.ShapeDtypeStruct((DIM, BATCH), jnp.float32),
        scratch_types=[
            pltpu.VMEM((DIM,), jnp.int32),
            pltpu.VMEM((DIM,), jnp.int32),
            pltpu.VMEM((G, BATCH), jnp.float32),
            pltpu.VMEM((G, BATCH), jnp.float32),
            pltpu.VMEM((G, BATCH), jnp.float32),
            pltpu.SemaphoreType.DMA,
            pltpu.SemaphoreType.DMA,
            pltpu.SemaphoreType.DMA,
            pltpu.SemaphoreType.DMA,
            pltpu.SemaphoreType.DMA,
            pltpu.SemaphoreType.DMA,
        ],
        compiler_params=pltpu.CompilerParams(needs_layout_passes=False),
    )
    def k(in_hbm, sel_hbm, out_hbm, sel_v, c_v, b0, b1, b2,
          g0, g1, g2, w0, w1, w2):
        cid = lax.axis_index("c")
        sid = lax.axis_index("s")
        wid = sid * 2 + cid
        nvalid = jnp.where(wid < NGROUPS % NUM_WORKERS, MAXP, MAXP - 1)

        pltpu.sync_copy(sel_hbm, sel_v)

        @plsc.parallel_loop(0, NSEG, unroll=8)
        def _(j):
            off = pl.multiple_of(j * LANES, LANES)
            seg = sel_v[pl.ds(off, LANES)]
            c_v[pl.ds(off, LANES)] = plsc.load_gather(sel_v, [seg])

        def fire_gather(p, buf, gsem):
            grp = wid + NUM_WORKERS * p
            idx = c_v.at[pl.ds(pl.multiple_of(grp * G, G), G)]
            pltpu.async_copy(in_hbm.at[idx], buf, gsem)

        fire_gather(0, b0, g0)
        fire_gather(1, b1, g1)

        bufs = ((b0, g0, w0), (b1, g1, w1), (b2, g2, w2))

        def step(t, carry):
            for par in range(3):
                buf, gsem, wsem = bufs[par]
                obuf, ogsem, owsem = bufs[(par + 2) % 3]
                p = 3 * t + par
                grp = wid + NUM_WORKERS * p

                @pl.when((p + 2 < nvalid) & (p >= 1))
                def _():
                    pltpu.make_async_copy(obuf, out_hbm.at[pl.ds(0, G)],
                                          owsem).wait()

                @pl.when(p + 2 < nvalid)
                def _():
                    fire_gather(p + 2, obuf, ogsem)

                @pl.when(p < nvalid)
                def _():
                    pltpu.make_async_copy(in_hbm.at[pl.ds(0, G)], buf,
                                          gsem).wait()
                    pltpu.async_copy(
                        buf, out_hbm.at[pl.ds(pl.multiple_of(grp * G, G), G)],
                        wsem)

            return carry

        lax.fori_loop(0, TSTEPS, step, 0)

        pltpu.make_async_copy(b0, out_hbm.at[pl.ds(0, G)], w0).wait()
        pltpu.make_async_copy(b1, out_hbm.at[pl.ds(0, G)], w1).wait()
        pltpu.make_async_copy(b2, out_hbm.at[pl.ds(0, G)], w2).wait()

    return k(inp_t, sel)


def kernel(input, forward_indices, backward_indices, shifts):
    ident = jnp.arange(DIM, dtype=jnp.int32)
    sel = jnp.where(
        shifts > 0,
        forward_indices,
        jnp.where(shifts < 0, backward_indices, ident),
    )
    out_t = _sc_permute(jnp.transpose(input), sel)
    return jnp.transpose(out_t)
